# baseline (speedup 1.0000x reference)
import functools

import jax
import jax.numpy as jnp
from jax import lax
from jax.experimental import pallas as pl
from jax.experimental.pallas import tpu as pltpu
from jax.experimental.pallas import tpu_sc as plsc

N = 10000
E = 320000
D = 128
C = 32
ROWS = 1000

NC = 2
NS = 16
NW = NC * NS
NP = 10240
RPS = NP // NS
PER_W = E // NW
K = 80
NCHUNK = PER_W // K


def _lin1_body(x_ref, w_ref, asrc_ref, adst_ref, x1_ref, aa_ref, ab_ref):
    x1 = jnp.dot(x_ref[...], w_ref[...], preferred_element_type=jnp.float32)
    x1_ref[...] = x1
    aa_ref[...] = jnp.dot(x1, asrc_ref[...], preferred_element_type=jnp.float32)
    ab_ref[...] = jnp.dot(x1, adst_ref[...], preferred_element_type=jnp.float32)


def _lin1(features, W1, att_src, att_dst):
    grid = N // ROWS
    return pl.pallas_call(
        _lin1_body,
        grid=(grid,),
        in_specs=[
            pl.BlockSpec((ROWS, D), lambda i: (i, 0)),
            pl.BlockSpec((D, D), lambda i: (0, 0)),
            pl.BlockSpec((D, 1), lambda i: (0, 0)),
            pl.BlockSpec((D, 1), lambda i: (0, 0)),
        ],
        out_specs=[
            pl.BlockSpec((ROWS, D), lambda i: (i, 0)),
            pl.BlockSpec((ROWS, 1), lambda i: (i, 0)),
            pl.BlockSpec((ROWS, 1), lambda i: (i, 0)),
        ],
        out_shape=[
            jax.ShapeDtypeStruct((N, D), jnp.float32),
            jax.ShapeDtypeStruct((N, 1), jnp.float32),
            jax.ShapeDtypeStruct((N, 1), jnp.float32),
        ],
    )(features, W1, att_src, att_dst)


def _elu(x):
    return jnp.where(x > 0, x, jnp.exp(jnp.minimum(x, 0.0)) - 1.0)


def _mix1_body(acc_ref, s_ref, w2_ref, h2_ref):
    s = s_ref[:, 0:1] + s_ref[:, 1:2] + 1e-16
    h1 = _elu((acc_ref[0] + acc_ref[1]) / s)
    h2_ref[...] = jnp.dot(h1, w2_ref[...], preferred_element_type=jnp.float32)


def _mix1(acc1, s_t, W2):
    rows = 1024
    grid = NP // rows
    return pl.pallas_call(
        _mix1_body,
        grid=(grid,),
        in_specs=[
            pl.BlockSpec((2, rows, D), lambda i: (0, i, 0)),
            pl.BlockSpec((rows, 2), lambda i: (i, 0)),
            pl.BlockSpec((D, C), lambda i: (0, 0)),
        ],
        out_specs=pl.BlockSpec((rows, C), lambda i: (i, 0)),
        out_shape=jax.ShapeDtypeStruct((NP, C), jnp.float32),
    )(acc1, s_t, W2)


def _mix2_body(accg_ref, s_ref, w2_ref, w1_ref, h4_ref):
    s = s_ref[:, 0:1] + s_ref[:, 1:2] + 1e-16
    g = (accg_ref[0] + accg_ref[1]) / s
    h3 = _elu(lax.dot_general(g, w2_ref[...], (((1,), (1,)), ((), ())),
                              preferred_element_type=jnp.float32))
    h4_ref[...] = lax.dot_general(h3, w1_ref[...], (((1,), (1,)), ((), ())),
                                  preferred_element_type=jnp.float32)


def _mix2(accg, s_t, W2, W1):
    rows = 1024
    grid = NP // rows
    return pl.pallas_call(
        _mix2_body,
        grid=(grid,),
        in_specs=[
            pl.BlockSpec((2, rows, C), lambda i: (0, i, 0)),
            pl.BlockSpec((rows, 2), lambda i: (i, 0)),
            pl.BlockSpec((D, C), lambda i: (0, 0)),
            pl.BlockSpec((D, D), lambda i: (0, 0)),
        ],
        out_specs=pl.BlockSpec((rows, D), lambda i: (i, 0)),
        out_shape=jax.ShapeDtypeStruct((NP, D), jnp.float32),
    )(accg, s_t, W2, W1)


_SC_MESH = plsc.VectorSubcoreMesh(
    core_axis_name="c", subcore_axis_name="s", num_cores=NC, num_subcores=NS)


def _bcast_lane(vec, lane):
    idx = jnp.zeros((16, 1), jnp.int32) + lane
    dn = lax.GatherDimensionNumbers(
        offset_dims=(), collapsed_slice_dims=(0,), start_index_map=(0,))
    return lax.gather(vec, idx, dn, (1,),
                      mode=lax.GatherScatterMode.PROMISE_IN_BOUNDS)


def _zero_chunks(buf_ref, nrows, width16):
    z = jnp.zeros((16,), jnp.float32)

    def zrow(i, _):
        for j in range(width16):
            buf_ref[i, pl.ds(j * 16, 16)] = z
        return 0

    lax.fori_loop(0, nrows, zrow, 0)


def _make_spmm(width, k, with_s):
    nchunk = PER_W // k
    assert nchunk % 2 == 1 and k % 16 == 0 and PER_W % k == 0

    out_type = [jax.ShapeDtypeStruct((NC, NP, width), jnp.float32)]
    scratch = [pltpu.VMEM_SHARED((NP, width), jnp.float32)]
    if with_s:
        out_type.append(jax.ShapeDtypeStruct((NC * NP,), jnp.float32))
        scratch.append(pltpu.VMEM_SHARED((NP,), jnp.float32))
        scratch.append(pltpu.VMEM((RPS,), jnp.float32))
    for _ in range(2):
        scratch += [
            pltpu.VMEM((k,), jnp.int32),
            pltpu.VMEM((k,), jnp.int32),
            pltpu.VMEM((k,), jnp.float32),
            pltpu.VMEM((k,), jnp.float32),
            pltpu.VMEM((k,), jnp.float32),
            pltpu.VMEM((k, width), jnp.float32),
        ]
    scratch += [pltpu.SemaphoreType.DMA, pltpu.SemaphoreType.DMA,
                pltpu.SemaphoreType.DMA]

    def body(*refs):
        it = iter(refs)
        x_hbm, src_hbm, dst_hbm, asrc_hbm, adst_hbm = (next(it) for _ in range(5))
        out_hbm = next(it)
        s_hbm = next(it) if with_s else None
        acc = next(it)
        ssum = next(it) if with_s else None
        bounce_v = next(it) if with_s else None
        bufs = [tuple(next(it) for _ in range(6)) for _ in range(2)]
        sem, sem_ab, sem_sc = next(it), next(it), next(it)

        c = lax.axis_index("c")
        s = lax.axis_index("s")
        wid = s * NC + c
        zbase = s * RPS

        rows0 = bufs[0][5]
        _zero_chunks(rows0, k, width // 16)
        for off in range(0, RPS, k):
            sz = min(k, RPS - off)
            pltpu.sync_copy(rows0.at[pl.ds(0, sz)],
                            acc.at[pl.ds(zbase + off, sz)])
        if with_s:
            def zvec(i, _):
                bounce_v[pl.ds(i * 16, 16)] = jnp.zeros((16,), jnp.float32)
                return 0
            lax.fori_loop(0, RPS // 16, zvec, 0)
            pltpu.sync_copy(bounce_v, ssum.at[pl.ds(zbase, RPS)])
        plsc.subcore_barrier()

        def load_idx(ci, b):
            sv, dv = b[0], b[1]
            base = pl.multiple_of(wid * PER_W + ci * k, 16)
            pltpu.sync_copy(src_hbm.at[pl.ds(base, k)], sv)
            pltpu.sync_copy(dst_hbm.at[pl.ds(base, k)], dv)

        def issue(b):
            sv, dv, av, bv, _, rv = b
            pltpu.async_copy(x_hbm.at[sv], rv, sem)
            pltpu.async_copy(asrc_hbm.at[sv], av, sem_ab)
            pltpu.async_copy(adst_hbm.at[dv], bv, sem_ab)

        def process(b):
            sv, dv, av, bv, ev, rv = b
            pltpu.make_async_copy(asrc_hbm.at[sv], av, sem_ab).wait()
            pltpu.make_async_copy(adst_hbm.at[dv], bv, sem_ab).wait()

            def vec(j, _):
                sl = pl.ds(j * 16, 16)
                logit = av[sl] + bv[sl]
                sig = 1.0 / (1.0 + jnp.exp(-logit))
                ev[sl] = jnp.exp(sig)
                return 0

            lax.fori_loop(0, k // 16, vec, 0)
            if with_s:
                pltpu.sync_copy(ev, ssum.at[dv], add=True)
            pltpu.make_async_copy(x_hbm.at[sv], rv, sem).wait()

        def scale(b):
            _, dv, _, _, ev, rv = b

            def group(g, _):
                ev16 = ev[pl.ds(g * 16, 16)]
                for ln in range(16):
                    eb = _bcast_lane(ev16, ln)
                    kk = g * 16 + ln
                    for j in range(width // 16):
                        sl = pl.ds(j * 16, 16)
                        rv[kk, sl] = rv[kk, sl] * eb
                return 0

            lax.fori_loop(0, k // 16, group, 0)

        def scatter_async(b):
            _, dv, _, _, _, rv = b
            pltpu.async_copy(rv, acc.at[dv], sem_sc, add=True)

        def wait_sc(b):
            _, dv, _, _, _, rv = b
            pltpu.make_async_copy(rv, acc.at[dv], sem_sc).wait()

        def step(ci, cur, nxt, first=False):
            process(cur)
            if not first:
                wait_sc(nxt)
            load_idx(ci + 1, nxt)
            issue(nxt)
            scale(cur)
            scatter_async(cur)

        load_idx(0, bufs[0])
        issue(bufs[0])
        step(0, bufs[0], bufs[1], first=True)

        def pair(p, _):
            ci0 = 2 * p + 1
            step(ci0, bufs[1], bufs[0])
            step(ci0 + 1, bufs[0], bufs[1])
            return 0

        lax.fori_loop(0, (nchunk - 3) // 2, pair, 0)
        step(nchunk - 2, bufs[1], bufs[0])
        process(bufs[0])
        wait_sc(bufs[1])
        scale(bufs[0])
        scatter_async(bufs[0])
        wait_sc(bufs[0])

        plsc.subcore_barrier()
        for off in range(0, RPS, k):
            sz = min(k, RPS - off)
            pltpu.sync_copy(acc.at[pl.ds(zbase + off, sz)],
                            out_hbm.at[c, pl.ds(zbase + off, sz)])
        if with_s:
            pltpu.sync_copy(ssum.at[pl.ds(zbase, RPS)], bounce_v)
            pltpu.sync_copy(
                bounce_v,
                s_hbm.at[pl.ds(pl.multiple_of(c * NP + zbase, 16), RPS)])

    params = (None if width % 128 == 0
              else pltpu.CompilerParams(use_tc_tiling_on_sc=False))
    return pl.kernel(body, out_type=out_type, mesh=_SC_MESH,
                     scratch_types=scratch, compiler_params=params)


_spmm1_sc = _make_spmm(D, 80, True)
_spmm2_sc = _make_spmm(C, 400, False)


def kernel(features, edge_index, W1, att_src1, att_dst1, W2):
    src = edge_index[0]
    dst = edge_index[1]
    att_src = att_src1.reshape(D, 1)
    att_dst = att_dst1.reshape(D, 1)

    x1, asrc, adst = _lin1(features, W1, att_src, att_dst)
    asrc = asrc.reshape(N)
    adst = adst.reshape(N)
    acc1, s_parts = _spmm1_sc(x1, src, dst, asrc, adst)
    s_t = s_parts.reshape(NC, NP).T
    h2 = _mix1(acc1, s_t, W2)
    accg, = _spmm2_sc(h2, src, dst, asrc, adst)
    h4 = _mix2(accg, s_t, W2, W1)
    return (h2[:N], h4[:N])


# speedup vs baseline: 1.0020x; 1.0020x over previous
import functools

import jax
import jax.numpy as jnp
from jax import lax
from jax.experimental import pallas as pl
from jax.experimental.pallas import tpu as pltpu
from jax.experimental.pallas import tpu_sc as plsc

N = 10000
E = 320000
D = 128
C = 32
ROWS = 1000

NC = 2
NS = 16
NW = NC * NS
NP = 10240
RPS = NP // NS
PER_W = E // NW
K = 80
NCHUNK = PER_W // K


def _lin1_body(x_ref, w_ref, asrc_ref, adst_ref, x1_ref, aa_ref, ab_ref):
    x1 = jnp.dot(x_ref[...], w_ref[...], preferred_element_type=jnp.float32)
    x1_ref[...] = x1
    aa_ref[...] = jnp.dot(x1, asrc_ref[...], preferred_element_type=jnp.float32)
    ab_ref[...] = jnp.dot(x1, adst_ref[...], preferred_element_type=jnp.float32)


def _lin1(features, W1, att_src, att_dst):
    grid = N // ROWS
    return pl.pallas_call(
        _lin1_body,
        grid=(grid,),
        in_specs=[
            pl.BlockSpec((ROWS, D), lambda i: (i, 0)),
            pl.BlockSpec((D, D), lambda i: (0, 0)),
            pl.BlockSpec((D, 1), lambda i: (0, 0)),
            pl.BlockSpec((D, 1), lambda i: (0, 0)),
        ],
        out_specs=[
            pl.BlockSpec((ROWS, D), lambda i: (i, 0)),
            pl.BlockSpec((ROWS, 1), lambda i: (i, 0)),
            pl.BlockSpec((ROWS, 1), lambda i: (i, 0)),
        ],
        out_shape=[
            jax.ShapeDtypeStruct((N, D), jnp.float32),
            jax.ShapeDtypeStruct((N, 1), jnp.float32),
            jax.ShapeDtypeStruct((N, 1), jnp.float32),
        ],
    )(features, W1, att_src, att_dst)


def _elu(x):
    return jnp.where(x > 0, x, jnp.exp(jnp.minimum(x, 0.0)) - 1.0)


def _mix1_body(acc_ref, s_ref, w2_ref, h2_ref):
    s = s_ref[:, 0:1] + s_ref[:, 1:2] + 1e-16
    h1 = _elu((acc_ref[0] + acc_ref[1]) / s)
    h2_ref[...] = jnp.dot(h1, w2_ref[...], preferred_element_type=jnp.float32)


def _mix1(acc1, s_t, W2):
    rows = 1024
    grid = NP // rows
    return pl.pallas_call(
        _mix1_body,
        grid=(grid,),
        in_specs=[
            pl.BlockSpec((2, rows, D), lambda i: (0, i, 0)),
            pl.BlockSpec((rows, 2), lambda i: (i, 0)),
            pl.BlockSpec((D, C), lambda i: (0, 0)),
        ],
        out_specs=pl.BlockSpec((rows, C), lambda i: (i, 0)),
        out_shape=jax.ShapeDtypeStruct((NP, C), jnp.float32),
    )(acc1, s_t, W2)


def _mix2_body(accg_ref, s_ref, w2_ref, w1_ref, h4_ref):
    s = s_ref[:, 0:1] + s_ref[:, 1:2] + 1e-16
    g = (accg_ref[0] + accg_ref[1]) / s
    h3 = _elu(lax.dot_general(g, w2_ref[...], (((1,), (1,)), ((), ())),
                              preferred_element_type=jnp.float32))
    h4_ref[...] = lax.dot_general(h3, w1_ref[...], (((1,), (1,)), ((), ())),
                                  preferred_element_type=jnp.float32)


def _mix2(accg, s_t, W2, W1):
    rows = 1024
    grid = NP // rows
    return pl.pallas_call(
        _mix2_body,
        grid=(grid,),
        in_specs=[
            pl.BlockSpec((2, rows, C), lambda i: (0, i, 0)),
            pl.BlockSpec((rows, 2), lambda i: (i, 0)),
            pl.BlockSpec((D, C), lambda i: (0, 0)),
            pl.BlockSpec((D, D), lambda i: (0, 0)),
        ],
        out_specs=pl.BlockSpec((rows, D), lambda i: (i, 0)),
        out_shape=jax.ShapeDtypeStruct((NP, D), jnp.float32),
    )(accg, s_t, W2, W1)


_SC_MESH = plsc.VectorSubcoreMesh(
    core_axis_name="c", subcore_axis_name="s", num_cores=NC, num_subcores=NS)


def _bcast_lane(vec, lane):
    idx = jnp.zeros((16, 1), jnp.int32) + lane
    dn = lax.GatherDimensionNumbers(
        offset_dims=(), collapsed_slice_dims=(0,), start_index_map=(0,))
    return lax.gather(vec, idx, dn, (1,),
                      mode=lax.GatherScatterMode.PROMISE_IN_BOUNDS)


def _zero_chunks(buf_ref, nrows, width16):
    z = jnp.zeros((16,), jnp.float32)

    def zrow(i, _):
        for j in range(width16):
            buf_ref[i, pl.ds(j * 16, 16)] = z
        return 0

    lax.fori_loop(0, nrows, zrow, 0)


def _make_spmm(width, k, with_s):
    nchunk = PER_W // k
    assert nchunk % 2 == 1 and k % 16 == 0 and PER_W % k == 0

    out_type = [jax.ShapeDtypeStruct((NC, NP, width), jnp.float32)]
    scratch = [pltpu.VMEM_SHARED((NP, width), jnp.float32)]
    if with_s:
        out_type.append(jax.ShapeDtypeStruct((NC * NP,), jnp.float32))
        scratch.append(pltpu.VMEM_SHARED((NP,), jnp.float32))
        scratch.append(pltpu.VMEM((RPS,), jnp.float32))
    for _ in range(3):
        scratch += [
            pltpu.VMEM((k,), jnp.int32),
            pltpu.VMEM((k,), jnp.int32),
            pltpu.VMEM((k,), jnp.float32),
            pltpu.VMEM((k,), jnp.float32),
            pltpu.VMEM((k,), jnp.float32),
            pltpu.VMEM((k, width), jnp.float32),
        ]
    scratch += [pltpu.SemaphoreType.DMA, pltpu.SemaphoreType.DMA,
                pltpu.SemaphoreType.DMA, pltpu.SemaphoreType.DMA]

    def body(*refs):
        it = iter(refs)
        x_hbm, src_hbm, dst_hbm, asrc_hbm, adst_hbm = (next(it) for _ in range(5))
        out_hbm = next(it)
        s_hbm = next(it) if with_s else None
        acc = next(it)
        ssum = next(it) if with_s else None
        bounce_v = next(it) if with_s else None
        bufs = [tuple(next(it) for _ in range(6)) for _ in range(3)]
        sem, sem_ab, sem_sc, sem_ss = (next(it) for _ in range(4))

        c = lax.axis_index("c")
        s = lax.axis_index("s")
        wid = s * NC + c
        zbase = s * RPS

        rows0 = bufs[0][5]
        _zero_chunks(rows0, k, width // 16)
        for off in range(0, RPS, k):
            sz = min(k, RPS - off)
            pltpu.sync_copy(rows0.at[pl.ds(0, sz)],
                            acc.at[pl.ds(zbase + off, sz)])
        if with_s:
            def zvec(i, _):
                bounce_v[pl.ds(i * 16, 16)] = jnp.zeros((16,), jnp.float32)
                return 0
            lax.fori_loop(0, RPS // 16, zvec, 0)
            pltpu.sync_copy(bounce_v, ssum.at[pl.ds(zbase, RPS)])
        plsc.subcore_barrier()

        def load_idx(ci, b):
            sv, dv = b[0], b[1]
            base = pl.multiple_of(wid * PER_W + ci * k, 16)
            pltpu.sync_copy(src_hbm.at[pl.ds(base, k)], sv)
            pltpu.sync_copy(dst_hbm.at[pl.ds(base, k)], dv)

        def issue(b):
            sv, dv, av, bv, _, rv = b
            pltpu.async_copy(x_hbm.at[sv], rv, sem)
            pltpu.async_copy(asrc_hbm.at[sv], av, sem_ab)
            pltpu.async_copy(adst_hbm.at[dv], bv, sem_ab)

        def process(b):
            sv, dv, av, bv, ev, rv = b
            pltpu.make_async_copy(asrc_hbm.at[sv], av, sem_ab).wait()
            pltpu.make_async_copy(adst_hbm.at[dv], bv, sem_ab).wait()

            def vec(j, _):
                sl = pl.ds(j * 16, 16)
                logit = av[sl] + bv[sl]
                sig = 1.0 / (1.0 + jnp.exp(-logit))
                ev[sl] = jnp.exp(sig)
                return 0

            lax.fori_loop(0, k // 16, vec, 0)
            if with_s:
                pltpu.async_copy(ev, ssum.at[dv], sem_ss, add=True)
            pltpu.make_async_copy(x_hbm.at[sv], rv, sem).wait()

        def scale(b):
            _, dv, _, _, ev, rv = b

            def group(g, _):
                ev16 = ev[pl.ds(g * 16, 16)]
                for ln in range(16):
                    eb = _bcast_lane(ev16, ln)
                    kk = g * 16 + ln
                    for j in range(width // 16):
                        sl = pl.ds(j * 16, 16)
                        rv[kk, sl] = rv[kk, sl] * eb
                return 0

            lax.fori_loop(0, k // 16, group, 0)

        def scatter_async(b):
            _, dv, _, _, _, rv = b
            pltpu.async_copy(rv, acc.at[dv], sem_sc, add=True)

        def wait_sc(b):
            _, dv, _, _, _, rv = b
            pltpu.make_async_copy(rv, acc.at[dv], sem_sc).wait()

        def wait_ss(b):
            _, dv, _, _, ev, _ = b
            pltpu.make_async_copy(ev, ssum.at[dv], sem_ss).wait()

        def step(cur, nxt, ci, drain):
            process(cur)
            if drain:
                wait_sc(nxt)
                if with_s:
                    wait_ss(nxt)
            load_idx(ci + 1, nxt)
            issue(nxt)
            scale(cur)
            scatter_async(cur)

        def b_of(ci):
            return bufs[ci % 3]

        load_idx(0, bufs[0])
        issue(bufs[0])
        for ci in (0, 1):
            step(b_of(ci), b_of(ci + 1), ci, drain=False)

        step(b_of(2), b_of(3), 2, drain=True)
        t = (nchunk - 5) // 3
        rem_start = 3 + 3 * t

        def triple(p, _):
            ci0 = 3 * p + 3
            step(bufs[0], bufs[1], ci0, drain=True)
            step(bufs[1], bufs[2], ci0 + 1, drain=True)
            step(bufs[2], bufs[0], ci0 + 2, drain=True)
            return 0

        lax.fori_loop(0, t, triple, 0)
        for ci in range(rem_start, nchunk - 1):
            step(b_of(ci), b_of(ci + 1), ci, drain=True)

        last = b_of(nchunk - 1)
        process(last)
        scale(last)
        scatter_async(last)
        for j in range(3):
            wait_sc(bufs[(nchunk - 3 + j) % 3])
            if with_s:
                wait_ss(bufs[(nchunk - 3 + j) % 3])

        plsc.subcore_barrier()
        for off in range(0, RPS, k):
            sz = min(k, RPS - off)
            pltpu.sync_copy(acc.at[pl.ds(zbase + off, sz)],
                            out_hbm.at[c, pl.ds(zbase + off, sz)])
        if with_s:
            pltpu.sync_copy(ssum.at[pl.ds(zbase, RPS)], bounce_v)
            pltpu.sync_copy(
                bounce_v,
                s_hbm.at[pl.ds(pl.multiple_of(c * NP + zbase, 16), RPS)])

    params = (None if width % 128 == 0
              else pltpu.CompilerParams(use_tc_tiling_on_sc=False))
    return pl.kernel(body, out_type=out_type, mesh=_SC_MESH,
                     scratch_types=scratch, compiler_params=params)


_spmm1_sc = _make_spmm(D, 80, True)
_spmm2_sc = _make_spmm(C, 400, False)


def kernel(features, edge_index, W1, att_src1, att_dst1, W2):
    src = edge_index[0]
    dst = edge_index[1]
    att_src = att_src1.reshape(D, 1)
    att_dst = att_dst1.reshape(D, 1)

    x1, asrc, adst = _lin1(features, W1, att_src, att_dst)
    asrc = asrc.reshape(N)
    adst = adst.reshape(N)
    acc1, s_parts = _spmm1_sc(x1, src, dst, asrc, adst)
    s_t = s_parts.reshape(NC, NP).T
    h2 = _mix1(acc1, s_t, W2)
    accg, = _spmm2_sc(h2, src, dst, asrc, adst)
    h4 = _mix2(accg, s_t, W2, W1)
    return (h2[:N], h4[:N])


# speedup vs baseline: 1.0046x; 1.0026x over previous
import functools

import jax
import jax.numpy as jnp
from jax import lax
from jax.experimental import pallas as pl
from jax.experimental.pallas import tpu as pltpu
from jax.experimental.pallas import tpu_sc as plsc

N = 10000
E = 320000
D = 128
C = 32
ROWS = 1000

NC = 2
NS = 16
NW = NC * NS
NP = 10240
RPS = NP // NS
PER_W = E // NW
K = 80
NCHUNK = PER_W // K


def _lin1_body(x_ref, w_ref, asrc_ref, adst_ref, x1_ref, aa_ref, ab_ref):
    x1 = jnp.dot(x_ref[...], w_ref[...], preferred_element_type=jnp.float32)
    x1_ref[...] = x1
    aa_ref[...] = jnp.dot(x1, asrc_ref[...], preferred_element_type=jnp.float32)
    ab_ref[...] = jnp.dot(x1, adst_ref[...], preferred_element_type=jnp.float32)


def _lin1(features, W1, att_src, att_dst):
    grid = N // ROWS
    return pl.pallas_call(
        _lin1_body,
        grid=(grid,),
        in_specs=[
            pl.BlockSpec((ROWS, D), lambda i: (i, 0)),
            pl.BlockSpec((D, D), lambda i: (0, 0)),
            pl.BlockSpec((D, 1), lambda i: (0, 0)),
            pl.BlockSpec((D, 1), lambda i: (0, 0)),
        ],
        out_specs=[
            pl.BlockSpec((ROWS, D), lambda i: (i, 0)),
            pl.BlockSpec((ROWS, 1), lambda i: (i, 0)),
            pl.BlockSpec((ROWS, 1), lambda i: (i, 0)),
        ],
        out_shape=[
            jax.ShapeDtypeStruct((N, D), jnp.float32),
            jax.ShapeDtypeStruct((N, 1), jnp.float32),
            jax.ShapeDtypeStruct((N, 1), jnp.float32),
        ],
    )(features, W1, att_src, att_dst)


def _elu(x):
    return jnp.where(x > 0, x, jnp.exp(jnp.minimum(x, 0.0)) - 1.0)


def _mix1_body(acc_ref, s_ref, w2_ref, h2_ref):
    s = s_ref[:, 0:1] + s_ref[:, 1:2] + 1e-16
    h1 = _elu((acc_ref[0] + acc_ref[1]) / s)
    h2_ref[...] = jnp.dot(h1, w2_ref[...], preferred_element_type=jnp.float32)


def _mix1(acc1, s_t, W2):
    rows = 1024
    grid = NP // rows
    return pl.pallas_call(
        _mix1_body,
        grid=(grid,),
        in_specs=[
            pl.BlockSpec((2, rows, D), lambda i: (0, i, 0)),
            pl.BlockSpec((rows, 2), lambda i: (i, 0)),
            pl.BlockSpec((D, C), lambda i: (0, 0)),
        ],
        out_specs=pl.BlockSpec((rows, C), lambda i: (i, 0)),
        out_shape=jax.ShapeDtypeStruct((NP, C), jnp.float32),
    )(acc1, s_t, W2)


def _mix2_body(accg_ref, s_ref, w2_ref, w1_ref, h4_ref):
    s = s_ref[:, 0:1] + s_ref[:, 1:2] + 1e-16
    g = (accg_ref[0] + accg_ref[1]) / s
    h3 = _elu(lax.dot_general(g, w2_ref[...], (((1,), (1,)), ((), ())),
                              preferred_element_type=jnp.float32))
    h4_ref[...] = lax.dot_general(h3, w1_ref[...], (((1,), (1,)), ((), ())),
                                  preferred_element_type=jnp.float32)


def _mix2(accg, s_t, W2, W1):
    rows = 1024
    grid = NP // rows
    return pl.pallas_call(
        _mix2_body,
        grid=(grid,),
        in_specs=[
            pl.BlockSpec((2, rows, C), lambda i: (0, i, 0)),
            pl.BlockSpec((rows, 2), lambda i: (i, 0)),
            pl.BlockSpec((D, C), lambda i: (0, 0)),
            pl.BlockSpec((D, D), lambda i: (0, 0)),
        ],
        out_specs=pl.BlockSpec((rows, D), lambda i: (i, 0)),
        out_shape=jax.ShapeDtypeStruct((NP, D), jnp.float32),
    )(accg, s_t, W2, W1)


_SC_MESH = plsc.VectorSubcoreMesh(
    core_axis_name="c", subcore_axis_name="s", num_cores=NC, num_subcores=NS)


def _bcast_lane(vec, lane):
    idx = jnp.zeros((16, 1), jnp.int32) + lane
    dn = lax.GatherDimensionNumbers(
        offset_dims=(), collapsed_slice_dims=(0,), start_index_map=(0,))
    return lax.gather(vec, idx, dn, (1,),
                      mode=lax.GatherScatterMode.PROMISE_IN_BOUNDS)


def _zero_chunks(buf_ref, nrows, width16):
    z = jnp.zeros((16,), jnp.float32)

    def zrow(i, _):
        for j in range(width16):
            buf_ref[i, pl.ds(j * 16, 16)] = z
        return 0

    lax.fori_loop(0, nrows, zrow, 0)


def _make_spmm(width, k, with_s):
    nchunk = PER_W // k
    assert nchunk % 2 == 1 and k % 16 == 0 and PER_W % k == 0

    out_type = [jax.ShapeDtypeStruct((NC, NP, width), jnp.float32)]
    scratch = [pltpu.VMEM_SHARED((NP, width), jnp.float32)]
    if with_s:
        out_type.append(jax.ShapeDtypeStruct((NC * NP,), jnp.float32))
        scratch.append(pltpu.VMEM_SHARED((NP,), jnp.float32))
        scratch.append(pltpu.VMEM((RPS,), jnp.float32))
    for _ in range(2):
        scratch += [
            pltpu.VMEM((k,), jnp.int32),
            pltpu.VMEM((k,), jnp.int32),
            pltpu.VMEM((k,), jnp.float32),
            pltpu.VMEM((k,), jnp.float32),
            pltpu.VMEM((k,), jnp.float32),
            pltpu.VMEM((k, width), jnp.float32),
        ]
    scratch += [pltpu.SemaphoreType.DMA, pltpu.SemaphoreType.DMA,
                pltpu.SemaphoreType.DMA]

    def body(*refs):
        it = iter(refs)
        x_hbm, src_hbm, dst_hbm, asrc_hbm, adst_hbm = (next(it) for _ in range(5))
        out_hbm = next(it)
        s_hbm = next(it) if with_s else None
        acc = next(it)
        ssum = next(it) if with_s else None
        bounce_v = next(it) if with_s else None
        bufs = [tuple(next(it) for _ in range(6)) for _ in range(2)]
        sem, sem_ab, sem_ss = next(it), next(it), next(it)

        c = lax.axis_index("c")
        s = lax.axis_index("s")
        wid = s * NC + c
        zbase = s * RPS

        rows0 = bufs[0][5]
        _zero_chunks(rows0, k, width // 16)
        for off in range(0, RPS, k):
            sz = min(k, RPS - off)
            pltpu.sync_copy(rows0.at[pl.ds(0, sz)],
                            acc.at[pl.ds(zbase + off, sz)])
        if with_s:
            def zvec(i, _):
                bounce_v[pl.ds(i * 16, 16)] = jnp.zeros((16,), jnp.float32)
                return 0
            lax.fori_loop(0, RPS // 16, zvec, 0)
            pltpu.sync_copy(bounce_v, ssum.at[pl.ds(zbase, RPS)])
        plsc.subcore_barrier()

        def load_idx(ci, b):
            sv, dv = b[0], b[1]
            base = pl.multiple_of(wid * PER_W + ci * k, 16)
            pltpu.sync_copy(src_hbm.at[pl.ds(base, k)], sv)
            pltpu.sync_copy(dst_hbm.at[pl.ds(base, k)], dv)

        def issue(b):
            sv, dv, av, bv, _, rv = b
            pltpu.async_copy(x_hbm.at[sv], rv, sem)
            pltpu.async_copy(asrc_hbm.at[sv], av, sem_ab)
            pltpu.async_copy(adst_hbm.at[dv], bv, sem_ab)

        def process(b):
            sv, dv, av, bv, ev, rv = b
            pltpu.make_async_copy(asrc_hbm.at[sv], av, sem_ab).wait()
            pltpu.make_async_copy(adst_hbm.at[dv], bv, sem_ab).wait()

            def vec(j, _):
                sl = pl.ds(j * 16, 16)
                logit = av[sl] + bv[sl]
                sig = 1.0 / (1.0 + jnp.exp(-logit))
                ev[sl] = jnp.exp(sig)
                return 0

            lax.fori_loop(0, k // 16, vec, 0)
            if with_s:
                pltpu.async_copy(ev, ssum.at[dv], sem_ss, add=True)
            pltpu.make_async_copy(x_hbm.at[sv], rv, sem).wait()

        def scale_scatter(b):
            _, dv, _, _, ev, rv = b

            def group(g, _):
                ev16 = ev[pl.ds(g * 16, 16)]
                for ln in range(16):
                    eb = _bcast_lane(ev16, ln)
                    kk = g * 16 + ln
                    for j in range(width // 16):
                        sl = pl.ds(j * 16, 16)
                        rv[kk, sl] = rv[kk, sl] * eb
                return 0

            lax.fori_loop(0, k // 16, group, 0)
            pltpu.sync_copy(rv, acc.at[dv], add=True)

        def wait_ss(b):
            _, dv, _, _, ev, _ = b
            pltpu.make_async_copy(ev, ssum.at[dv], sem_ss).wait()

        def step(ci, cur, nxt, drain=True):
            process(cur)
            if with_s and drain:
                wait_ss(nxt)
            load_idx(ci + 1, nxt)
            issue(nxt)
            scale_scatter(cur)

        load_idx(0, bufs[0])
        issue(bufs[0])
        step(0, bufs[0], bufs[1], drain=False)
        step(1, bufs[1], bufs[0])

        def pair(p, _):
            ci0 = 2 * p + 2
            step(ci0, bufs[0], bufs[1])
            step(ci0 + 1, bufs[1], bufs[0])
            return 0

        lax.fori_loop(0, (nchunk - 3) // 2, pair, 0)
        process(bufs[0])
        scale_scatter(bufs[0])
        if with_s:
            wait_ss(bufs[1])
            wait_ss(bufs[0])

        plsc.subcore_barrier()
        for off in range(0, RPS, k):
            sz = min(k, RPS - off)
            pltpu.sync_copy(acc.at[pl.ds(zbase + off, sz)],
                            out_hbm.at[c, pl.ds(zbase + off, sz)])
        if with_s:
            pltpu.sync_copy(ssum.at[pl.ds(zbase, RPS)], bounce_v)
            pltpu.sync_copy(
                bounce_v,
                s_hbm.at[pl.ds(pl.multiple_of(c * NP + zbase, 16), RPS)])

    params = (None if width % 128 == 0
              else pltpu.CompilerParams(use_tc_tiling_on_sc=False))
    return pl.kernel(body, out_type=out_type, mesh=_SC_MESH,
                     scratch_types=scratch, compiler_params=params)


_spmm1_sc = _make_spmm(D, 80, True)
_spmm2_sc = _make_spmm(C, 400, False)


def kernel(features, edge_index, W1, att_src1, att_dst1, W2):
    src = edge_index[0]
    dst = edge_index[1]
    att_src = att_src1.reshape(D, 1)
    att_dst = att_dst1.reshape(D, 1)

    x1, asrc, adst = _lin1(features, W1, att_src, att_dst)
    asrc = asrc.reshape(N)
    adst = adst.reshape(N)
    acc1, s_parts = _spmm1_sc(x1, src, dst, asrc, adst)
    s_t = s_parts.reshape(NC, NP).T
    h2 = _mix1(acc1, s_t, W2)
    accg, = _spmm2_sc(h2, src, dst, asrc, adst)
    h4 = _mix2(accg, s_t, W2, W1)
    return (h2[:N], h4[:N])


# speedup vs baseline: 1.0982x; 1.0932x over previous
import functools

import jax
import jax.numpy as jnp
from jax import lax
from jax.experimental import pallas as pl
from jax.experimental.pallas import tpu as pltpu
from jax.experimental.pallas import tpu_sc as plsc

N = 10000
E = 320000
D = 128
C = 32
ROWS = 1000

NC = 2
NS = 16
NW = NC * NS
NP = 10240
RPS = NP // NS
PER_W = E // NW
K = 80
NCHUNK = PER_W // K


def _lin1_body(x_ref, w_ref, asrc_ref, adst_ref, x1_ref, aa_ref, ab_ref):
    x1 = jnp.dot(x_ref[...], w_ref[...], preferred_element_type=jnp.float32)
    x1_ref[...] = x1
    aa_ref[...] = jnp.dot(x1, asrc_ref[...], preferred_element_type=jnp.float32)
    ab_ref[...] = jnp.dot(x1, adst_ref[...], preferred_element_type=jnp.float32)


def _lin1(features, W1, att_src, att_dst):
    grid = N // ROWS
    return pl.pallas_call(
        _lin1_body,
        grid=(grid,),
        in_specs=[
            pl.BlockSpec((ROWS, D), lambda i: (i, 0)),
            pl.BlockSpec((D, D), lambda i: (0, 0)),
            pl.BlockSpec((D, 1), lambda i: (0, 0)),
            pl.BlockSpec((D, 1), lambda i: (0, 0)),
        ],
        out_specs=[
            pl.BlockSpec((ROWS, D), lambda i: (i, 0)),
            pl.BlockSpec((ROWS, 1), lambda i: (i, 0)),
            pl.BlockSpec((ROWS, 1), lambda i: (i, 0)),
        ],
        out_shape=[
            jax.ShapeDtypeStruct((N, D), jnp.float32),
            jax.ShapeDtypeStruct((N, 1), jnp.float32),
            jax.ShapeDtypeStruct((N, 1), jnp.float32),
        ],
    )(features, W1, att_src, att_dst)


def _elu(x):
    return jnp.where(x > 0, x, jnp.exp(jnp.minimum(x, 0.0)) - 1.0)


def _mix1_body(acc_ref, s_ref, w2_ref, h2_ref):
    s = s_ref[:, 0:1] + s_ref[:, 1:2] + 1e-16
    h1 = _elu((acc_ref[0] + acc_ref[1]) / s)
    h2_ref[...] = jnp.dot(h1, w2_ref[...], preferred_element_type=jnp.float32)


def _mix1(acc1, s_t, W2):
    rows = 2000
    grid = N // rows
    return pl.pallas_call(
        _mix1_body,
        grid=(grid,),
        in_specs=[
            pl.BlockSpec((2, rows, D), lambda i: (0, i, 0)),
            pl.BlockSpec((rows, 2), lambda i: (i, 0)),
            pl.BlockSpec((D, C), lambda i: (0, 0)),
        ],
        out_specs=pl.BlockSpec((rows, C), lambda i: (i, 0)),
        out_shape=jax.ShapeDtypeStruct((N, C), jnp.float32),
    )(acc1, s_t, W2)


def _mix2_body(accg_ref, s_ref, w2_ref, w1_ref, h4_ref):
    s = s_ref[:, 0:1] + s_ref[:, 1:2] + 1e-16
    g = (accg_ref[0] + accg_ref[1]) / s
    h3 = _elu(lax.dot_general(g, w2_ref[...], (((1,), (1,)), ((), ())),
                              preferred_element_type=jnp.float32))
    h4_ref[...] = lax.dot_general(h3, w1_ref[...], (((1,), (1,)), ((), ())),
                                  preferred_element_type=jnp.float32)


def _mix2(accg, s_t, W2, W1):
    rows = 2000
    grid = N // rows
    return pl.pallas_call(
        _mix2_body,
        grid=(grid,),
        in_specs=[
            pl.BlockSpec((2, rows, C), lambda i: (0, i, 0)),
            pl.BlockSpec((rows, 2), lambda i: (i, 0)),
            pl.BlockSpec((D, C), lambda i: (0, 0)),
            pl.BlockSpec((D, D), lambda i: (0, 0)),
        ],
        out_specs=pl.BlockSpec((rows, D), lambda i: (i, 0)),
        out_shape=jax.ShapeDtypeStruct((N, D), jnp.float32),
    )(accg, s_t, W2, W1)


_SC_MESH = plsc.VectorSubcoreMesh(
    core_axis_name="c", subcore_axis_name="s", num_cores=NC, num_subcores=NS)


def _bcast_lane(vec, lane):
    idx = jnp.zeros((16, 1), jnp.int32) + lane
    dn = lax.GatherDimensionNumbers(
        offset_dims=(), collapsed_slice_dims=(0,), start_index_map=(0,))
    return lax.gather(vec, idx, dn, (1,),
                      mode=lax.GatherScatterMode.PROMISE_IN_BOUNDS)


def _zero_chunks(buf_ref, nrows, width16):
    z = jnp.zeros((16,), jnp.float32)

    def zrow(i, _):
        for j in range(width16):
            buf_ref[i, pl.ds(j * 16, 16)] = z
        return 0

    lax.fori_loop(0, nrows, zrow, 0)


def _make_spmm(width, k, with_s):
    nchunk = PER_W // k
    assert nchunk % 2 == 1 and k % 16 == 0 and PER_W % k == 0

    out_type = [jax.ShapeDtypeStruct((NC, NP, width), jnp.float32)]
    scratch = [pltpu.VMEM_SHARED((NP, width), jnp.float32)]
    if with_s:
        out_type.append(jax.ShapeDtypeStruct((NC * NP,), jnp.float32))
        scratch.append(pltpu.VMEM_SHARED((NP,), jnp.float32))
        scratch.append(pltpu.VMEM((RPS,), jnp.float32))
    for _ in range(2):
        scratch += [
            pltpu.VMEM((k,), jnp.int32),
            pltpu.VMEM((k,), jnp.int32),
            pltpu.VMEM((k,), jnp.float32),
            pltpu.VMEM((k,), jnp.float32),
            pltpu.VMEM((k,), jnp.float32),
            pltpu.VMEM((k, width), jnp.float32),
        ]
    scratch += [pltpu.SemaphoreType.DMA, pltpu.SemaphoreType.DMA]

    def body(*refs):
        it = iter(refs)
        x_hbm, src_hbm, dst_hbm, asrc_hbm, adst_hbm = (next(it) for _ in range(5))
        out_hbm = next(it)
        s_hbm = next(it) if with_s else None
        acc = next(it)
        ssum = next(it) if with_s else None
        bounce_v = next(it) if with_s else None
        bufs = [tuple(next(it) for _ in range(6)) for _ in range(2)]
        sem, sem_ab = next(it), next(it)

        c = lax.axis_index("c")
        s = lax.axis_index("s")
        wid = s * NC + c
        zbase = s * RPS

        rows0 = bufs[0][5]
        _zero_chunks(rows0, k, width // 16)
        for off in range(0, RPS, k):
            sz = min(k, RPS - off)
            pltpu.sync_copy(rows0.at[pl.ds(0, sz)],
                            acc.at[pl.ds(zbase + off, sz)])
        if with_s:
            def zvec(i, _):
                bounce_v[pl.ds(i * 16, 16)] = jnp.zeros((16,), jnp.float32)
                return 0
            lax.fori_loop(0, RPS // 16, zvec, 0)
            pltpu.sync_copy(bounce_v, ssum.at[pl.ds(zbase, RPS)])
        plsc.subcore_barrier()

        def load_idx(ci, b):
            sv, dv = b[0], b[1]
            base = pl.multiple_of(wid * PER_W + ci * k, 16)
            pltpu.sync_copy(src_hbm.at[pl.ds(base, k)], sv)
            pltpu.sync_copy(dst_hbm.at[pl.ds(base, k)], dv)

        def issue(b):
            sv, dv, av, bv, _, rv = b
            pltpu.async_copy(x_hbm.at[sv], rv, sem)
            pltpu.async_copy(asrc_hbm.at[sv], av, sem_ab)
            pltpu.async_copy(adst_hbm.at[dv], bv, sem_ab)

        def process(b):
            sv, dv, av, bv, ev, rv = b
            pltpu.make_async_copy(asrc_hbm.at[sv], av, sem_ab).wait()
            pltpu.make_async_copy(adst_hbm.at[dv], bv, sem_ab).wait()

            def vec(j, _):
                sl = pl.ds(j * 16, 16)
                logit = av[sl] + bv[sl]
                sig = 1.0 / (1.0 + jnp.exp(-logit))
                ev[sl] = jnp.exp(sig)
                return 0

            lax.fori_loop(0, k // 16, vec, 0)
            if with_s:
                pltpu.sync_copy(ev, ssum.at[dv], add=True)
            pltpu.make_async_copy(x_hbm.at[sv], rv, sem).wait()

        def scale_scatter(b):
            _, dv, _, _, ev, rv = b

            def group(g, _):
                ev16 = ev[pl.ds(g * 16, 16)]
                for ln in range(16):
                    eb = _bcast_lane(ev16, ln)
                    kk = g * 16 + ln
                    for j in range(width // 16):
                        sl = pl.ds(j * 16, 16)
                        rv[kk, sl] = rv[kk, sl] * eb
                return 0

            lax.fori_loop(0, k // 16, group, 0)
            pltpu.sync_copy(rv, acc.at[dv], add=True)

        def step(ci, cur, nxt):
            load_idx(ci + 1, nxt)
            process(cur)
            issue(nxt)
            scale_scatter(cur)

        load_idx(0, bufs[0])
        issue(bufs[0])

        def pair(p, _):
            ci0 = 2 * p
            step(ci0, bufs[0], bufs[1])
            step(ci0 + 1, bufs[1], bufs[0])
            return 0

        lax.fori_loop(0, (nchunk - 1) // 2, pair, 0)
        process(bufs[0])
        scale_scatter(bufs[0])

        plsc.subcore_barrier()
        for off in range(0, RPS, k):
            sz = min(k, RPS - off)
            pltpu.sync_copy(acc.at[pl.ds(zbase + off, sz)],
                            out_hbm.at[c, pl.ds(zbase + off, sz)])
        if with_s:
            pltpu.sync_copy(ssum.at[pl.ds(zbase, RPS)], bounce_v)
            pltpu.sync_copy(
                bounce_v,
                s_hbm.at[pl.ds(pl.multiple_of(c * NP + zbase, 16), RPS)])

    params = (None if width % 128 == 0
              else pltpu.CompilerParams(use_tc_tiling_on_sc=False))
    return pl.kernel(body, out_type=out_type, mesh=_SC_MESH,
                     scratch_types=scratch, compiler_params=params)


_spmm1_sc = _make_spmm(D, 80, True)
_spmm2_sc = _make_spmm(C, 400, False)


def kernel(features, edge_index, W1, att_src1, att_dst1, W2):
    src = edge_index[0]
    dst = edge_index[1]
    att_src = att_src1.reshape(D, 1)
    att_dst = att_dst1.reshape(D, 1)

    x1, asrc, adst = _lin1(features, W1, att_src, att_dst)
    asrc = asrc.reshape(N)
    adst = adst.reshape(N)
    acc1, s_parts = _spmm1_sc(x1, src, dst, asrc, adst)
    s_t = s_parts.reshape(NC, NP).T
    h2 = _mix1(acc1, s_t, W2)
    accg, = _spmm2_sc(h2, src, dst, asrc, adst)
    h4 = _mix2(accg, s_t, W2, W1)
    return (h2, h4)


# speedup vs baseline: 1.1064x; 1.0074x over previous
import functools

import jax
import jax.numpy as jnp
from jax import lax
from jax.experimental import pallas as pl
from jax.experimental.pallas import tpu as pltpu
from jax.experimental.pallas import tpu_sc as plsc

N = 10000
E = 320000
D = 128
C = 32
ROWS = 1000

NC = 2
NS = 16
NW = NC * NS
NP = 10240
RPS = NP // NS
PER_W = E // NW
K = 80
NCHUNK = PER_W // K


def _lin1_body(x_ref, w_ref, asrc_ref, adst_ref, x1_ref, aa_ref, ab_ref):
    x1 = jnp.dot(x_ref[...], w_ref[...], preferred_element_type=jnp.float32)
    x1_ref[...] = x1
    aa_ref[...] = jnp.dot(x1, asrc_ref[...], preferred_element_type=jnp.float32)
    ab_ref[...] = jnp.dot(x1, adst_ref[...], preferred_element_type=jnp.float32)


def _lin1(features, W1, att_src, att_dst):
    grid = N // ROWS
    return pl.pallas_call(
        _lin1_body,
        grid=(grid,),
        in_specs=[
            pl.BlockSpec((ROWS, D), lambda i: (i, 0)),
            pl.BlockSpec((D, D), lambda i: (0, 0)),
            pl.BlockSpec((D, 1), lambda i: (0, 0)),
            pl.BlockSpec((D, 1), lambda i: (0, 0)),
        ],
        out_specs=[
            pl.BlockSpec((ROWS, D), lambda i: (i, 0)),
            pl.BlockSpec((ROWS, 1), lambda i: (i, 0)),
            pl.BlockSpec((ROWS, 1), lambda i: (i, 0)),
        ],
        out_shape=[
            jax.ShapeDtypeStruct((N, D), jnp.float32),
            jax.ShapeDtypeStruct((N, 1), jnp.float32),
            jax.ShapeDtypeStruct((N, 1), jnp.float32),
        ],
    )(features, W1, att_src, att_dst)


def _elu(x):
    return jnp.where(x > 0, x, jnp.exp(jnp.minimum(x, 0.0)) - 1.0)


def _mix1_body(acc_ref, s_ref, w2_ref, h2_ref):
    s = s_ref[:, 0:1] + s_ref[:, 1:2] + 1e-16
    h1 = _elu((acc_ref[0] + acc_ref[1]) / s)
    h2_ref[...] = jnp.dot(h1, w2_ref[...], preferred_element_type=jnp.float32)


def _mix1(acc1, s_t, W2):
    rows = 2000
    grid = N // rows
    return pl.pallas_call(
        _mix1_body,
        grid=(grid,),
        in_specs=[
            pl.BlockSpec((2, rows, D), lambda i: (0, i, 0)),
            pl.BlockSpec((rows, 2), lambda i: (i, 0)),
            pl.BlockSpec((D, C), lambda i: (0, 0)),
        ],
        out_specs=pl.BlockSpec((rows, C), lambda i: (i, 0)),
        out_shape=jax.ShapeDtypeStruct((N, C), jnp.float32),
    )(acc1, s_t, W2)


def _mix2_body(accg_ref, s_ref, w2_ref, w1_ref, h4_ref):
    s = s_ref[:, 0:1] + s_ref[:, 1:2] + 1e-16
    g = (accg_ref[0] + accg_ref[1]) / s
    h3 = _elu(lax.dot_general(g, w2_ref[...], (((1,), (1,)), ((), ())),
                              preferred_element_type=jnp.float32))
    h4_ref[...] = lax.dot_general(h3, w1_ref[...], (((1,), (1,)), ((), ())),
                                  preferred_element_type=jnp.float32)


def _mix2(accg, s_t, W2, W1):
    rows = 2000
    grid = N // rows
    return pl.pallas_call(
        _mix2_body,
        grid=(grid,),
        in_specs=[
            pl.BlockSpec((2, rows, C), lambda i: (0, i, 0)),
            pl.BlockSpec((rows, 2), lambda i: (i, 0)),
            pl.BlockSpec((D, C), lambda i: (0, 0)),
            pl.BlockSpec((D, D), lambda i: (0, 0)),
        ],
        out_specs=pl.BlockSpec((rows, D), lambda i: (i, 0)),
        out_shape=jax.ShapeDtypeStruct((N, D), jnp.float32),
    )(accg, s_t, W2, W1)


_SC_MESH = plsc.VectorSubcoreMesh(
    core_axis_name="c", subcore_axis_name="s", num_cores=NC, num_subcores=NS)


def _bcast_lane(vec, lane):
    idx = jnp.zeros((16, 1), jnp.int32) + lane
    dn = lax.GatherDimensionNumbers(
        offset_dims=(), collapsed_slice_dims=(0,), start_index_map=(0,))
    return lax.gather(vec, idx, dn, (1,),
                      mode=lax.GatherScatterMode.PROMISE_IN_BOUNDS)


def _zero_chunks(buf_ref, nrows, width16):
    z = jnp.zeros((16,), jnp.float32)

    def zrow(i, _):
        for j in range(width16):
            buf_ref[i, pl.ds(j * 16, 16)] = z
        return 0

    lax.fori_loop(0, nrows, zrow, 0)


def _make_spmm(width, k, with_s):
    nchunk = PER_W // k
    assert nchunk % 2 == 1 and k % 16 == 0 and PER_W % k == 0

    out_type = [jax.ShapeDtypeStruct((NC, NP, width), jnp.float32)]
    scratch = [pltpu.VMEM_SHARED((NP, width), jnp.float32)]
    if with_s:
        out_type.append(jax.ShapeDtypeStruct((NC * NP,), jnp.float32))
        scratch.append(pltpu.VMEM_SHARED((NP,), jnp.float32))
        scratch.append(pltpu.VMEM((RPS,), jnp.float32))
    for _ in range(2):
        scratch += [
            pltpu.VMEM((k,), jnp.int32),
            pltpu.VMEM((k,), jnp.int32),
            pltpu.VMEM((k,), jnp.float32),
            pltpu.VMEM((k,), jnp.float32),
            pltpu.VMEM((k,), jnp.float32),
            pltpu.VMEM((k, width), jnp.float32),
        ]
    scratch += [
        pltpu.VMEM_SHARED((N,), jnp.float32),
        pltpu.VMEM_SHARED((N,), jnp.float32),
        pltpu.VMEM((624,), jnp.float32),
        pltpu.SemaphoreType.DMA, pltpu.SemaphoreType.DMA,
    ]

    def body(*refs):
        it = iter(refs)
        x_hbm, src_hbm, dst_hbm, asrc_hbm, adst_hbm = (next(it) for _ in range(5))
        out_hbm = next(it)
        s_hbm = next(it) if with_s else None
        acc = next(it)
        ssum = next(it) if with_s else None
        bounce_v = next(it) if with_s else None
        bufs = [tuple(next(it) for _ in range(6)) for _ in range(2)]
        asp, bsp, stage_v = next(it), next(it), next(it)
        sem, sem_ab = next(it), next(it)

        c = lax.axis_index("c")
        s = lax.axis_index("s")
        wid = s * NC + c
        zbase = s * RPS

        rows0 = bufs[0][5]
        _zero_chunks(rows0, k, width // 16)
        for off in range(0, RPS, k):
            sz = min(k, RPS - off)
            pltpu.sync_copy(rows0.at[pl.ds(0, sz)],
                            acc.at[pl.ds(zbase + off, sz)])
        if with_s:
            def zvec(i, _):
                bounce_v[pl.ds(i * 16, 16)] = jnp.zeros((16,), jnp.float32)
                return 0
            lax.fori_loop(0, RPS // 16, zvec, 0)
            pltpu.sync_copy(bounce_v, ssum.at[pl.ds(zbase, RPS)])

        sb = pl.multiple_of(s * 624, 16)
        for src_arr, dst_arr in ((asrc_hbm, asp), (adst_hbm, bsp)):
            pltpu.sync_copy(src_arr.at[pl.ds(sb, 624)], stage_v)
            pltpu.sync_copy(stage_v, dst_arr.at[pl.ds(sb, 624)])

        @pl.when(s == 0)
        def _tail():
            for src_arr, dst_arr in ((asrc_hbm, asp), (adst_hbm, bsp)):
                pltpu.sync_copy(src_arr.at[pl.ds(9984, 16)],
                                stage_v.at[pl.ds(0, 16)])
                pltpu.sync_copy(stage_v.at[pl.ds(0, 16)],
                                dst_arr.at[pl.ds(9984, 16)])
        plsc.subcore_barrier()

        def load_idx(ci, b):
            sv, dv = b[0], b[1]
            base = pl.multiple_of(wid * PER_W + ci * k, 16)
            pltpu.sync_copy(src_hbm.at[pl.ds(base, k)], sv)
            pltpu.sync_copy(dst_hbm.at[pl.ds(base, k)], dv)

        def issue(b):
            sv, dv, av, bv, _, rv = b
            pltpu.async_copy(x_hbm.at[sv], rv, sem)
            pltpu.async_copy(asp.at[sv], av, sem_ab)
            pltpu.async_copy(bsp.at[dv], bv, sem_ab)

        def process(b):
            sv, dv, av, bv, ev, rv = b
            pltpu.make_async_copy(asp.at[sv], av, sem_ab).wait()
            pltpu.make_async_copy(bsp.at[dv], bv, sem_ab).wait()

            def vec(j, _):
                sl = pl.ds(j * 16, 16)
                logit = av[sl] + bv[sl]
                sig = 1.0 / (1.0 + jnp.exp(-logit))
                ev[sl] = jnp.exp(sig)
                return 0

            lax.fori_loop(0, k // 16, vec, 0)
            if with_s:
                pltpu.sync_copy(ev, ssum.at[dv], add=True)
            pltpu.make_async_copy(x_hbm.at[sv], rv, sem).wait()

        def scale_scatter(b):
            _, dv, _, _, ev, rv = b

            def group(g, _):
                ev16 = ev[pl.ds(g * 16, 16)]
                for ln in range(16):
                    eb = _bcast_lane(ev16, ln)
                    kk = g * 16 + ln
                    for j in range(width // 16):
                        sl = pl.ds(j * 16, 16)
                        rv[kk, sl] = rv[kk, sl] * eb
                return 0

            lax.fori_loop(0, k // 16, group, 0)
            pltpu.sync_copy(rv, acc.at[dv], add=True)

        def step(ci, cur, nxt):
            load_idx(ci + 1, nxt)
            process(cur)
            issue(nxt)
            scale_scatter(cur)

        load_idx(0, bufs[0])
        issue(bufs[0])

        def pair(p, _):
            ci0 = 2 * p
            step(ci0, bufs[0], bufs[1])
            step(ci0 + 1, bufs[1], bufs[0])
            return 0

        lax.fori_loop(0, (nchunk - 1) // 2, pair, 0)
        process(bufs[0])
        scale_scatter(bufs[0])

        plsc.subcore_barrier()
        for off in range(0, RPS, k):
            sz = min(k, RPS - off)
            pltpu.sync_copy(acc.at[pl.ds(zbase + off, sz)],
                            out_hbm.at[c, pl.ds(zbase + off, sz)])
        if with_s:
            pltpu.sync_copy(ssum.at[pl.ds(zbase, RPS)], bounce_v)
            pltpu.sync_copy(
                bounce_v,
                s_hbm.at[pl.ds(pl.multiple_of(c * NP + zbase, 16), RPS)])

    params = (None if width % 128 == 0
              else pltpu.CompilerParams(use_tc_tiling_on_sc=False))
    return pl.kernel(body, out_type=out_type, mesh=_SC_MESH,
                     scratch_types=scratch, compiler_params=params)


_spmm1_sc = _make_spmm(D, 80, True)
_spmm2_sc = _make_spmm(C, 400, False)


def kernel(features, edge_index, W1, att_src1, att_dst1, W2):
    src = edge_index[0]
    dst = edge_index[1]
    att_src = att_src1.reshape(D, 1)
    att_dst = att_dst1.reshape(D, 1)

    x1, asrc, adst = _lin1(features, W1, att_src, att_dst)
    asrc = asrc.reshape(N)
    adst = adst.reshape(N)
    acc1, s_parts = _spmm1_sc(x1, src, dst, asrc, adst)
    s_t = s_parts.reshape(NC, NP).T
    h2 = _mix1(acc1, s_t, W2)
    accg, = _spmm2_sc(h2, src, dst, asrc, adst)
    h4 = _mix2(accg, s_t, W2, W1)
    return (h2, h4)
